# baseline (speedup 1.0000x reference)
import functools

import jax
import jax.numpy as jnp
from jax import lax
from jax.experimental import pallas as pl
from jax.experimental.pallas import tpu as pltpu
from jax.experimental.pallas import tpu_sc as plsc

NF = 26
V = 100000
D = 16
B = 16384
C = 13
OUTW = C + NF * D

NC, NS, L = 2, 16, 16
NW = NC * NS
CPW = NF * D // NW
QTR = B // 4
GL = QTR // L

SPLIT = 49920
H2SZ = 50048
TOFF = SPLIT + H2SZ
LUTW = TOFF + 128


def _embed_body(contT_hbm, catT_hbm, tblT_hbm, tail_hbm, outT_hbm,
                lut_v, idx_v, out_a, out_b, s_h1, s_h2, s_idx, s_w0, s_w1):
    wid = lax.axis_index("s") * NC + lax.axis_index("c")

    @pl.when(wid < C)
    def _():
        for q in range(4):
            pltpu.sync_copy(contT_hbm.at[wid, pl.ds(q * QTR, QTR)], out_a)
            pltpu.sync_copy(out_a, outT_hbm.at[wid, pl.ds(q * QTR, QTR)])

    def h1_cp(f, d):
        return pltpu.make_async_copy(
            tblT_hbm.at[f, d, pl.ds(0, SPLIT)],
            lut_v.at[pl.ds(0, SPLIT)], s_h1)

    def h2_cps(f, d):
        return (
            pltpu.make_async_copy(
                tblT_hbm.at[f, d, pl.ds(SPLIT, H2SZ)],
                lut_v.at[pl.ds(SPLIT, H2SZ)], s_h2),
            pltpu.make_async_copy(
                tail_hbm.at[f, d], lut_v.at[pl.ds(TOFF, 128)], s_h2),
        )

    lanes = lax.iota(jnp.int32, L)
    splitv = jnp.full((L,), SPLIT, jnp.int32)

    bufs = (out_a, out_b)
    wsems = (s_w0, s_w1)
    npend = [0, 0]
    for j in range(CPW):
        r = wid * CPW + j
        f = lax.div(r, D)
        d = lax.rem(r, D)
        idx_cp = pltpu.make_async_copy(catT_hbm.at[f], idx_v, s_idx)
        if j == 0:
            idx_cp.start()
            idx_cp.wait()
            h1_cp(f, d).start()
            for cp in h2_cps(f, d):
                cp.start()
        else:
            @pl.when(d == 0)
            def _():
                idx_cp.start()
                idx_cp.wait()
        if j + 1 < CPW:
            rn = r + 1
            fn = lax.div(rn, D)
            dn = lax.rem(rn, D)
        h1_cp(f, d).wait()
        for q in range(4):
            p = q % 2
            buf = bufs[p]
            if npend[p]:
                pltpu.make_async_copy(
                    buf, outT_hbm.at[C + r, pl.ds(q * QTR, QTR)],
                    wsems[p]).wait()
                npend[p] -= 1

            def pass1(i):
                s = pl.ds(i * L, L)
                iv = idx_v[pl.ds(q * QTR + i * L, L)]
                buf[s] = plsc.load_gather(lut_v, [iv], mask=iv < splitv)
            pl.loop(0, GL, unroll=4)(pass1)
            if q == 3 and j + 1 < CPW:
                h1_cp(fn, dn).start()
            if q == 0:
                for cp in h2_cps(f, d):
                    cp.wait()

            def pass2(i):
                s = pl.ds(i * L, L)
                iv = idx_v[pl.ds(q * QTR + i * L, L)]
                m = iv < splitv
                g2 = plsc.load_gather(lut_v, [iv], mask=jnp.logical_not(m))
                buf[s] = jnp.where(m, buf[s], g2)
            pl.loop(0, GL, unroll=4)(pass2)
            pltpu.make_async_copy(
                buf, outT_hbm.at[C + r, pl.ds(q * QTR, QTR)],
                wsems[p]).start()
            npend[p] += 1
        if j + 1 < CPW:
            for cp in h2_cps(fn, dn):
                cp.start()
    r_last = wid * CPW + CPW - 1
    for p in range(2):
        while npend[p]:
            pltpu.make_async_copy(
                bufs[p], outT_hbm.at[C + r_last, pl.ds(p * QTR, QTR)],
                wsems[p]).wait()
            npend[p] -= 1


@jax.jit
def _embed(contT, catT, tblT, tailp):
    mesh = plsc.VectorSubcoreMesh(core_axis_name="c", subcore_axis_name="s")
    return pl.kernel(
        _embed_body,
        out_type=jax.ShapeDtypeStruct((OUTW, B), jnp.float32),
        mesh=mesh,
        compiler_params=pltpu.CompilerParams(
            use_tc_tiling_on_sc=True, needs_layout_passes=False),
        scratch_types=[
            pltpu.VMEM((LUTW,), jnp.float32),
            pltpu.VMEM((B,), jnp.int32),
            pltpu.VMEM((QTR,), jnp.float32),
            pltpu.VMEM((QTR,), jnp.float32),
            pltpu.SemaphoreType.DMA,
            pltpu.SemaphoreType.DMA,
            pltpu.SemaphoreType.DMA,
            pltpu.SemaphoreType.DMA,
            pltpu.SemaphoreType.DMA,
        ],
    )(contT, catT, tblT, tailp)


def kernel(continuous, categorical, tables):
    contT = continuous.T
    catT = categorical.T
    tblT = jnp.transpose(tables, (0, 2, 1))
    tailp = jnp.pad(tblT[:, :, TOFF:], ((0, 0), (0, 0), (0, 96)))
    outT = _embed(contT, catT, tblT, tailp)
    return outT.T


# speedup vs baseline: 2.8537x; 2.8537x over previous
import functools

import jax
import jax.numpy as jnp
from jax import lax
from jax.experimental import pallas as pl
from jax.experimental.pallas import tpu as pltpu
from jax.experimental.pallas import tpu_sc as plsc

NF = 26
V = 100000
D = 16
B = 16384
C = 13
OUTW = C + NF * D

NC, NS, L = 2, 16, 16
NW = NC * NS
CPW = NF * D // NW
HALF = B // 2
QTR = B // 4
GL = QTR // L


def _embed_body(contT_hbm, catT_hbm, tblT_hbm, outT_hbm,
                lut_v, idx_v, out_a, out_b, s_lut, s_idx, s_w0, s_w1):
    wid = lax.axis_index("s") * NC + lax.axis_index("c")

    @pl.when(wid < C)
    def _():
        for q in range(4):
            pltpu.sync_copy(contT_hbm.at[wid, pl.ds(q * QTR, QTR)], out_a)
            pltpu.sync_copy(out_a, outT_hbm.at[wid, pl.ds(q * QTR, QTR)])

    bufs = (out_a, out_b)
    wsems = (s_w0, s_w1)
    npend = [0, 0]
    for j in range(CPW):
        r = wid * CPW + j
        f = lax.div(r, D)
        d = lax.rem(r, D)
        lut_cp = pltpu.make_async_copy(tblT_hbm.at[f, d], lut_v, s_lut)
        lut_cp.start()
        idx_cp = pltpu.make_async_copy(catT_hbm.at[f], idx_v, s_idx)
        if j == 0:
            idx_cp.start()
            idx_cp.wait()
        else:
            @pl.when(d == 0)
            def _():
                idx_cp.start()
                idx_cp.wait()
        lut_cp.wait()
        for q in range(4):
            p = q % 2
            buf = bufs[p]
            if npend[p]:
                pltpu.make_async_copy(
                    buf, outT_hbm.at[C + r, pl.ds(q * QTR, QTR)],
                    wsems[p]).wait()
                npend[p] -= 1

            def gather(i):
                s = pl.ds(i * L, L)
                iv = idx_v[pl.ds(q * QTR + i * L, L)]
                buf[s] = plsc.load_gather(lut_v, [iv])
            plsc.parallel_loop(0, GL, unroll=8)(gather)
            pltpu.make_async_copy(
                buf, outT_hbm.at[C + r, pl.ds(q * QTR, QTR)],
                wsems[p]).start()
            npend[p] += 1
    r_last = wid * CPW + CPW - 1
    for p in range(2):
        while npend[p]:
            pltpu.make_async_copy(
                bufs[p], outT_hbm.at[C + r_last, pl.ds(p * QTR, QTR)],
                wsems[p]).wait()
            npend[p] -= 1


@jax.jit
def _embed(contT, catT, tblT):
    mesh = plsc.VectorSubcoreMesh(core_axis_name="c", subcore_axis_name="s")
    return pl.kernel(
        _embed_body,
        out_type=jax.ShapeDtypeStruct((OUTW, B), jnp.float32),
        mesh=mesh,
        compiler_params=pltpu.CompilerParams(
            use_tc_tiling_on_sc=True, needs_layout_passes=False),
        scratch_types=[
            pltpu.VMEM((V,), jnp.float32),
            pltpu.VMEM((B,), jnp.int32),
            pltpu.VMEM((QTR,), jnp.float32),
            pltpu.VMEM((QTR,), jnp.float32),
            pltpu.SemaphoreType.DMA,
            pltpu.SemaphoreType.DMA,
            pltpu.SemaphoreType.DMA,
            pltpu.SemaphoreType.DMA,
        ],
    )(contT, catT, tblT)


def kernel(continuous, categorical, tables):
    contT = continuous.T
    catT = categorical.T
    tblT = jnp.transpose(tables, (0, 2, 1))
    outT = _embed(contT, catT, tblT)
    return outT.T
